# Initial kernel scaffold; baseline (speedup 1.0000x reference)
import jax
import jax.numpy as jnp
from jax.experimental import pallas as pl


def kernel(x, pos_table):
    raise NotImplementedError("write your pallas kernel here")


# speedup vs baseline: 1.9551x; 1.9551x over previous
import jax
import jax.numpy as jnp
from jax.experimental import pallas as pl


def _body(x_ref, pos_ref, o_ref):
    o_ref[...] = x_ref[...] + pos_ref[...][None]


def kernel(x, pos_table):
    B, S, D = x.shape
    S_BLK = 512
    return pl.pallas_call(
        _body,
        grid=(S // S_BLK,),
        in_specs=[
            pl.BlockSpec((B, S_BLK, D), lambda i: (0, i, 0)),
            pl.BlockSpec((S_BLK, D), lambda i: (i, 0)),
        ],
        out_specs=pl.BlockSpec((B, S_BLK, D), lambda i: (0, i, 0)),
        out_shape=jax.ShapeDtypeStruct(x.shape, x.dtype),
    )(x, pos_table)
